# baseline (speedup 1.0000x reference)
import functools

import jax
import jax.numpy as jnp
from jax import lax
from jax.experimental import pallas as pl
from jax.experimental.pallas import tpu as pltpu
from jax.experimental.pallas import tpu_sc as plsc

NC = 2
NS = 16
NW = NC * NS

BATCH = 4096
SEQ = 200
EMB = 64
TOTAL = BATCH * SEQ
PER_W = TOTAL // NW
CHUNK = 128
CHUNKS = PER_W // CHUNK
SUB = 4
BIG = CHUNK * SUB
NBIG = PER_W // BIG


def _make_gather():
  mesh = plsc.VectorSubcoreMesh(
      core_axis_name="c", subcore_axis_name="s",
      num_cores=NC, num_subcores=NS)

  @functools.partial(
      pl.kernel,
      out_type=jax.ShapeDtypeStruct((TOTAL, EMB), jnp.float32),
      mesh=mesh,
      scratch_types=[
          pltpu.VMEM((CHUNKS, CHUNK), jnp.int32),
          [pltpu.VMEM((BIG, EMB), jnp.float32)] * 2,
          [pltpu.SemaphoreType.DMA] * 2,
          [pltpu.SemaphoreType.DMA] * 2,
      ],
      compiler_params=pltpu.CompilerParams(use_tc_tiling_on_sc=False),
  )
  def gather_kernel(word_hbm, table_hbm, out_hbm, idx_v, bufs, gsems, ssems):
    wid = lax.axis_index("s") * NC + lax.axis_index("c")
    pltpu.sync_copy(word_hbm.at[wid], idx_v)

    def fill(g, b):
      for k in range(SUB):
        pltpu.async_copy(table_hbm.at[idx_v.at[g * SUB + k]],
                         bufs[b].at[pl.ds(k * CHUNK, CHUNK)], gsems[b])

    def drain_fill(g, b):
      for k in range(SUB):
        pltpu.make_async_copy(table_hbm.at[idx_v.at[g * SUB + k]],
                              bufs[b].at[pl.ds(k * CHUNK, CHUNK)],
                              gsems[b]).wait()

    def store(g, b):
      base = (wid * NBIG + g) * BIG
      return pltpu.async_copy(bufs[b], out_hbm.at[pl.ds(base, BIG)],
                              ssems[b])

    def wait_store(g, b):
      base = (wid * NBIG + g) * BIG
      pltpu.make_async_copy(bufs[b], out_hbm.at[pl.ds(base, BIG)],
                            ssems[b]).wait()

    fill(0, 0)

    def outer(g0, carry):
      for b in range(2):
        g = g0 + b
        nb = 1 - b

        @pl.when(g + 1 < NBIG)
        def _():
          @pl.when(g >= 1)
          def _():
            wait_store(g - 1, nb)
          fill(g + 1, nb)

        drain_fill(g, b)
        store(g, b)
      return carry

    lax.fori_loop(0, NBIG // 2, lambda i, c: outer(i * 2, c), 0)

    wait_store(NBIG - 2, 0)
    wait_store(NBIG - 1, 1)

  return gather_kernel


_gather = _make_gather()


def kernel(WORD, word_table):
  idx = WORD.reshape(NW, CHUNKS, CHUNK)
  out = _gather(idx, word_table)
  return out.reshape(TOTAL, EMB, 1)


# speedup vs baseline: 1.0002x; 1.0002x over previous
import functools

import jax
import jax.numpy as jnp
from jax import lax
from jax.experimental import pallas as pl
from jax.experimental.pallas import tpu as pltpu
from jax.experimental.pallas import tpu_sc as plsc

NC = 2
NS = 16
NW = NC * NS

BATCH = 4096
SEQ = 200
EMB = 64
TOTAL = BATCH * SEQ
PER_W = TOTAL // NW
CHUNK = 512
CHUNKS = PER_W // CHUNK
SUB = 1
BIG = CHUNK * SUB
NBIG = PER_W // BIG


def _make_gather():
  mesh = plsc.VectorSubcoreMesh(
      core_axis_name="c", subcore_axis_name="s",
      num_cores=NC, num_subcores=NS)

  @functools.partial(
      pl.kernel,
      out_type=jax.ShapeDtypeStruct((TOTAL, EMB), jnp.float32),
      mesh=mesh,
      scratch_types=[
          pltpu.VMEM((CHUNKS, CHUNK), jnp.int32),
          [pltpu.VMEM((BIG, EMB), jnp.float32)] * 2,
          [pltpu.SemaphoreType.DMA] * 2,
          [pltpu.SemaphoreType.DMA] * 2,
      ],
      compiler_params=pltpu.CompilerParams(use_tc_tiling_on_sc=False),
  )
  def gather_kernel(word_hbm, table_hbm, out_hbm, idx_v, bufs, gsems, ssems):
    wid = lax.axis_index("s") * NC + lax.axis_index("c")
    pltpu.sync_copy(word_hbm.at[wid], idx_v)

    def fill(g, b):
      for k in range(SUB):
        pltpu.async_copy(table_hbm.at[idx_v.at[g * SUB + k]],
                         bufs[b].at[pl.ds(k * CHUNK, CHUNK)], gsems[b])

    def drain_fill(g, b):
      for k in range(SUB):
        pltpu.make_async_copy(table_hbm.at[idx_v.at[g * SUB + k]],
                              bufs[b].at[pl.ds(k * CHUNK, CHUNK)],
                              gsems[b]).wait()

    def store(g, b):
      base = (wid * NBIG + g) * BIG
      return pltpu.async_copy(bufs[b], out_hbm.at[pl.ds(base, BIG)],
                              ssems[b])

    def wait_store(g, b):
      base = (wid * NBIG + g) * BIG
      pltpu.make_async_copy(bufs[b], out_hbm.at[pl.ds(base, BIG)],
                            ssems[b]).wait()

    fill(0, 0)

    def outer(g0, carry):
      for b in range(2):
        g = g0 + b
        nb = 1 - b

        @pl.when(g + 1 < NBIG)
        def _():
          @pl.when(g >= 1)
          def _():
            wait_store(g - 1, nb)
          fill(g + 1, nb)

        drain_fill(g, b)
        store(g, b)
      return carry

    lax.fori_loop(0, NBIG // 2, lambda i, c: outer(i * 2, c), 0)

    wait_store(NBIG - 2, 0)
    wait_store(NBIG - 1, 1)

  return gather_kernel


_gather = _make_gather()


def kernel(WORD, word_table):
  idx = WORD.reshape(NW, CHUNKS, CHUNK)
  out = _gather(idx, word_table)
  return out.reshape(TOTAL, EMB, 1)
